# Initial kernel scaffold; baseline (speedup 1.0000x reference)
import jax
import jax.numpy as jnp
from jax.experimental import pallas as pl


def kernel(x, edge_index, edge_attr, u, batch, W1, b1, W2, b2, W3, b3, W4, b4):
    raise NotImplementedError("write your pallas kernel here")


# speedup vs baseline: 1.7161x; 1.7161x over previous
import functools

import jax
import jax.numpy as jnp
from jax import lax
from jax.experimental import pallas as pl
from jax.experimental.pallas import tpu as pltpu
from jax.experimental.pallas import tpu_sc as plsc

N = 10000
NE = 160000
H = 512
NPAD = 10240
EPAD = 163840
NC, NS, L = 2, 16, 16
IB = 128
NCHUNK = 4

def _gather_body(x_hbm, idx_hbm, out_hbm, idx_v, buf_v, sem):
    c = lax.axis_index("c")
    s = lax.axis_index("s")
    wid = s * NC + c
    ew = EPAD // (NC * NS)
    nb = ew // IB
    pltpu.sync_copy(idx_hbm.at[pl.ds(wid * nb, nb)], idx_v)

    def body(j, _):
        pltpu.async_copy(x_hbm.at[idx_v.at[j]], buf_v, sem).wait()
        pltpu.sync_copy(buf_v, out_hbm.at[pl.ds(wid * ew + j * IB, IB)])
        return 0

    lax.fori_loop(0, nb, body, 0)


@functools.lru_cache(maxsize=None)
def _build_sc_gather():
    mesh = plsc.VectorSubcoreMesh(core_axis_name="c", subcore_axis_name="s",
                                  num_cores=NC, num_subcores=NS)
    return pl.kernel(
        _gather_body,
        out_type=jax.ShapeDtypeStruct((EPAD, 128), jnp.float32),
        mesh=mesh,
        scratch_types=[
            pltpu.VMEM((EPAD // (NC * NS) // IB, IB), jnp.int32),
            pltpu.VMEM((IB, 128), jnp.float32),
            pltpu.SemaphoreType.DMA,
        ],
    )


def _sc_gather(x_pad, col_pad):
    return _build_sc_gather()(x_pad, col_pad)


def _scatter_chunks(s, h_a, h_b, sum_a, sum_b, z_hbm, zc_hbm, cnt_hbm,
                    idx_v, upd_v, ones_v, acc, cacc, with_counts):
    nb = (EPAD // NS) // IB
    stripe = NPAD // NS

    def zero_acc():
        pltpu.sync_copy(z_hbm.at[pl.ds(s * stripe, stripe)],
                        acc.at[pl.ds(s * stripe, stripe)])

    def accumulate(h_ref, counts):
        def body(j, _):
            pltpu.sync_copy(h_ref.at[pl.ds(s * (EPAD // NS) + j * IB, IB)],
                            upd_v)
            pltpu.sync_copy(upd_v, acc.at[idx_v.at[j]], add=True)
            if counts:
                pltpu.sync_copy(ones_v, cacc.at[idx_v.at[j]], add=True)
            return 0

        lax.fori_loop(0, nb, body, 0)

    def flush(sum_ref):
        pltpu.sync_copy(acc.at[pl.ds(s * stripe, stripe)],
                        sum_ref.at[pl.ds(s * stripe, stripe)])

    zero_acc()
    if with_counts:
        @pl.when(s == 0)
        def _():
            pltpu.sync_copy(zc_hbm, cacc)
    plsc.subcore_barrier()
    accumulate(h_a, with_counts)
    plsc.subcore_barrier()
    flush(sum_a)
    if with_counts:
        @pl.when(s == 0)
        def _():
            pltpu.sync_copy(cacc, cnt_hbm)
    zero_acc()
    plsc.subcore_barrier()
    accumulate(h_b, False)
    plsc.subcore_barrier()
    flush(sum_b)


def _scatter_kernel_body(row_hbm, h0, h1, h2, h3, z_hbm, zc_hbm,
                         s0, s1, s2, s3, cnt_hbm,
                         idx_v, upd_v, ones_v, acc, cacc):
    c = lax.axis_index("c")
    s = lax.axis_index("s")
    nb = (EPAD // NS) // IB
    pltpu.sync_copy(row_hbm.at[pl.ds(s * nb, nb)], idx_v)

    def ones_init(i, _):
        ones_v[pl.ds(i * L, L)] = jnp.full((L,), 1.0, jnp.float32)
        return 0

    lax.fori_loop(0, IB // L, ones_init, 0)

    @pl.when(c == 0)
    def _():
        _scatter_chunks(s, h0, h1, s0, s1, z_hbm, zc_hbm, cnt_hbm,
                        idx_v, upd_v, ones_v, acc, cacc, True)

    @pl.when(c == 1)
    def _():
        _scatter_chunks(s, h2, h3, s2, s3, z_hbm, zc_hbm, cnt_hbm,
                        idx_v, upd_v, ones_v, acc, cacc, False)


@functools.lru_cache(maxsize=None)
def _build_sc_scatter():
    mesh = plsc.VectorSubcoreMesh(core_axis_name="c", subcore_axis_name="s",
                                  num_cores=NC, num_subcores=NS)
    return pl.kernel(
        _scatter_kernel_body,
        out_type=[jax.ShapeDtypeStruct((NPAD, 128), jnp.float32)] * NCHUNK
                 + [jax.ShapeDtypeStruct((NPAD,), jnp.float32)],
        mesh=mesh,
        scratch_types=[
            pltpu.VMEM(((EPAD // NS) // IB, IB), jnp.int32),
            pltpu.VMEM((IB, 128), jnp.float32),
            pltpu.VMEM((IB,), jnp.float32),
            pltpu.VMEM_SHARED((NPAD, 128), jnp.float32),
            pltpu.VMEM_SHARED((NPAD,), jnp.float32),
        ],
    )


def _sc_scatter(row_pad, h0, h1, h2, h3, zeros_h, zeros_c):
    return _build_sc_scatter()(row_pad, h0, h1, h2, h3, zeros_h, zeros_c)


BE = 256


def _edge_mlp_body(xg_ref, e_ref, w1a_ref, w1b_ref, b1_ref,
                   h0_ref, h1_ref, h2_ref, h3_ref):
    h = jnp.dot(xg_ref[...], w1a_ref[...], preferred_element_type=jnp.float32)
    h = h + jnp.dot(e_ref[...], w1b_ref[...],
                    preferred_element_type=jnp.float32)
    h = jnp.maximum(h + b1_ref[...], 0.0)
    h0_ref[...] = h[:, 0:128]
    h1_ref[...] = h[:, 128:256]
    h2_ref[...] = h[:, 256:384]
    h3_ref[...] = h[:, 384:512]


def _edge_mlp(xg, e, w1a, w1b, b1):
    grid = (NE // BE,)
    return pl.pallas_call(
        _edge_mlp_body,
        grid=grid,
        in_specs=[
            pl.BlockSpec((BE, 128), lambda i: (i, 0)),
            pl.BlockSpec((BE, H), lambda i: (i, 0)),
            pl.BlockSpec((128, H), lambda i: (0, 0)),
            pl.BlockSpec((H, H), lambda i: (0, 0)),
            pl.BlockSpec((1, H), lambda i: (0, 0)),
        ],
        out_specs=[pl.BlockSpec((BE, 128), lambda i: (i, 0))] * NCHUNK,
        out_shape=[jax.ShapeDtypeStruct((EPAD, 128), jnp.float32)] * NCHUNK,
    )(xg, e, w1a, w1b, b1)


BN = 256


def _node_mlp_body(xp_ref, s0_ref, s1_ref, s2_ref, s3_ref, cnt_ref, b2d_ref,
                   u_ref, w2_ref, b2_ref, w3a_ref, w3b_ref, w3c_ref, b3_ref,
                   w4_ref, b4_ref, o_ref):
    cnt = cnt_ref[...]
    m = jnp.maximum(cnt, 1.0)
    meanh = jnp.concatenate(
        [s0_ref[...], s1_ref[...], s2_ref[...], s3_ref[...]], axis=1) / m
    agg = jnp.dot(meanh, w2_ref[...], preferred_element_type=jnp.float32)
    agg = agg + jnp.where(cnt > 0.0, 1.0, 0.0) * b2_ref[...]
    hid = jnp.dot(xp_ref[...], w3a_ref[...],
                  preferred_element_type=jnp.float32)
    hid = hid + jnp.dot(agg, w3b_ref[...], preferred_element_type=jnp.float32)
    onehot = (b2d_ref[...] == lax.broadcasted_iota(
        jnp.int32, (BN, 64), 1)).astype(jnp.float32)
    ug = jnp.dot(onehot, u_ref[...], preferred_element_type=jnp.float32)
    hid = hid + jnp.dot(ug, w3c_ref[...], preferred_element_type=jnp.float32)
    hid = jnp.maximum(hid + b3_ref[...], 0.0)
    o_ref[...] = jnp.dot(hid, w4_ref[...],
                         preferred_element_type=jnp.float32) + b4_ref[...]


def _node_mlp(xp, s0, s1, s2, s3, cnt, b2d, u, w2, b2, w3a, w3b, w3c, b3,
              w4, b4):
    grid = (NPAD // BN,)
    return pl.pallas_call(
        _node_mlp_body,
        grid=grid,
        in_specs=[
            pl.BlockSpec((BN, 128), lambda i: (i, 0)),
            pl.BlockSpec((BN, 128), lambda i: (i, 0)),
            pl.BlockSpec((BN, 128), lambda i: (i, 0)),
            pl.BlockSpec((BN, 128), lambda i: (i, 0)),
            pl.BlockSpec((BN, 128), lambda i: (i, 0)),
            pl.BlockSpec((BN, 1), lambda i: (i, 0)),
            pl.BlockSpec((BN, 1), lambda i: (i, 0)),
            pl.BlockSpec((64, 16), lambda i: (0, 0)),
            pl.BlockSpec((H, H), lambda i: (0, 0)),
            pl.BlockSpec((1, H), lambda i: (0, 0)),
            pl.BlockSpec((128, H), lambda i: (0, 0)),
            pl.BlockSpec((H, H), lambda i: (0, 0)),
            pl.BlockSpec((16, H), lambda i: (0, 0)),
            pl.BlockSpec((1, H), lambda i: (0, 0)),
            pl.BlockSpec((H, 1), lambda i: (0, 0)),
            pl.BlockSpec((1, 1), lambda i: (0, 0)),
        ],
        out_specs=pl.BlockSpec((BN, 1), lambda i: (i, 0)),
        out_shape=jax.ShapeDtypeStruct((NPAD, 1), jnp.float32),
    )(xp, s0, s1, s2, s3, cnt, b2d, u, w2, b2, w3a, w3b, w3c, b3, w4, b4)


def kernel(x, edge_index, edge_attr, u, batch, W1, b1, W2, b2, W3, b3, W4,
           b4):
    f32 = jnp.float32
    row = edge_index[0].astype(jnp.int32)
    col = edge_index[1].astype(jnp.int32)

    x_pad = jnp.zeros((NPAD, 128), f32).at[:N, :9].set(x)
    col_pad = jnp.concatenate(
        [col, jnp.zeros((EPAD - NE,), jnp.int32)]).reshape(EPAD // IB, IB)
    sent = (jnp.arange(EPAD - NE, dtype=jnp.int32) % (NPAD - N)) + N
    row_pad = jnp.concatenate([row, sent]).reshape(EPAD // IB, IB)

    w1a = jnp.zeros((128, H), f32).at[:9].set(W1[:9])
    w1b = W1[9:]
    w3a = jnp.zeros((128, H), f32).at[:9].set(W3[:9])
    w3b = W3[9:9 + H]
    w3c = W3[9 + H:]
    batch2d = jnp.concatenate(
        [batch.astype(jnp.int32),
         jnp.zeros((NPAD - N,), jnp.int32)]).reshape(NPAD, 1)
    zeros_h = jnp.zeros((NPAD, 128), f32)
    zeros_c = jnp.zeros((NPAD,), f32)

    xg = _sc_gather(x_pad, col_pad)
    h0, h1, h2, h3 = _edge_mlp(xg, edge_attr, w1a, w1b, b1.reshape(1, H))
    s0, s1, s2, s3, cnt = _sc_scatter(row_pad, h0, h1, h2, h3, zeros_h,
                                      zeros_c)
    out = _node_mlp(x_pad, s0, s1, s2, s3, cnt.reshape(NPAD, 1), batch2d, u,
                    W2, b2.reshape(1, H), w3a, w3b, w3c, b3.reshape(1, H),
                    W4, b4.reshape(1, 1))
    return out[:N]


# speedup vs baseline: 2.4309x; 1.4165x over previous
import functools

import jax
import jax.numpy as jnp
from jax import lax
from jax.experimental import pallas as pl
from jax.experimental.pallas import tpu as pltpu
from jax.experimental.pallas import tpu_sc as plsc

N = 10000
NE = 160000
H = 512
NPAD = 10240
EPAD = 163840
NC, NS, L = 2, 16, 16
IB = 128
NCHUNK = 4

def _gather_body(x_hbm, idx_hbm, out_hbm, idx_v, buf_v, sem):
    c = lax.axis_index("c")
    s = lax.axis_index("s")
    wid = s * NC + c
    ew = EPAD // (NC * NS)
    nb = ew // IB
    pltpu.sync_copy(idx_hbm.at[pl.ds(wid * nb, nb)], idx_v)

    buf0, buf1 = buf_v.at[0], buf_v.at[1]
    sem0, sem1 = sem.at[0], sem.at[1]
    pltpu.async_copy(x_hbm.at[idx_v.at[0]], buf0, sem0)

    def body(jj, _):
        j = 2 * jj
        pltpu.async_copy(x_hbm.at[idx_v.at[j + 1]], buf1, sem1)
        pltpu.make_async_copy(x_hbm.at[idx_v.at[j]], buf0, sem0).wait()
        pltpu.sync_copy(buf0, out_hbm.at[pl.ds(wid * ew + j * IB, IB)])

        @pl.when(jj < nb // 2 - 1)
        def _():
            pltpu.async_copy(x_hbm.at[idx_v.at[j + 2]], buf0, sem0)

        pltpu.make_async_copy(x_hbm.at[idx_v.at[j + 1]], buf1, sem1).wait()
        pltpu.sync_copy(buf1, out_hbm.at[pl.ds(wid * ew + (j + 1) * IB, IB)])
        return 0

    lax.fori_loop(0, nb // 2, body, 0)


@functools.lru_cache(maxsize=None)
def _build_sc_gather():
    mesh = plsc.VectorSubcoreMesh(core_axis_name="c", subcore_axis_name="s",
                                  num_cores=NC, num_subcores=NS)
    return pl.kernel(
        _gather_body,
        out_type=jax.ShapeDtypeStruct((EPAD, 128), jnp.float32),
        mesh=mesh,
        scratch_types=[
            pltpu.VMEM((EPAD // (NC * NS) // IB, IB), jnp.int32),
            pltpu.VMEM((2, IB, 128), jnp.float32),
            pltpu.SemaphoreType.DMA((2,)),
        ],
    )


def _sc_gather(x_pad, col_pad):
    return _build_sc_gather()(x_pad, col_pad)


def _scatter_chunks(s, h_a, h_b, sum_a, sum_b, z_hbm, zc_hbm, cnt_hbm,
                    idx_v, upd_v, ones_v, acc, cacc, sem, with_counts):
    nb = (EPAD // NS) // IB
    stripe = NPAD // NS

    def zero_acc():
        pltpu.sync_copy(z_hbm.at[pl.ds(s * stripe, stripe)],
                        acc.at[pl.ds(s * stripe, stripe)])

    def accumulate(h_ref, counts):
        base = s * (EPAD // NS)
        upd0, upd1 = upd_v.at[0], upd_v.at[1]
        sem0, sem1 = sem.at[0], sem.at[1]
        pltpu.async_copy(h_ref.at[pl.ds(base, IB)], upd0, sem0)

        def body(jj, _):
            j = 2 * jj
            pltpu.async_copy(h_ref.at[pl.ds(base + (j + 1) * IB, IB)],
                             upd1, sem1)
            pltpu.make_async_copy(h_ref.at[pl.ds(base + j * IB, IB)],
                                  upd0, sem0).wait()
            pltpu.sync_copy(upd0, acc.at[idx_v.at[j]], add=True)
            if counts:
                pltpu.sync_copy(ones_v, cacc.at[idx_v.at[j]], add=True)

            @pl.when(jj < nb // 2 - 1)
            def _():
                pltpu.async_copy(h_ref.at[pl.ds(base + (j + 2) * IB, IB)],
                                 upd0, sem0)

            pltpu.make_async_copy(h_ref.at[pl.ds(base + (j + 1) * IB, IB)],
                                  upd1, sem1).wait()
            pltpu.sync_copy(upd1, acc.at[idx_v.at[j + 1]], add=True)
            if counts:
                pltpu.sync_copy(ones_v, cacc.at[idx_v.at[j + 1]], add=True)
            return 0

        lax.fori_loop(0, nb // 2, body, 0)

    def flush(sum_ref):
        pltpu.sync_copy(acc.at[pl.ds(s * stripe, stripe)],
                        sum_ref.at[pl.ds(s * stripe, stripe)])

    zero_acc()
    if with_counts:
        @pl.when(s == 0)
        def _():
            pltpu.sync_copy(zc_hbm, cacc)
    plsc.subcore_barrier()
    accumulate(h_a, with_counts)
    plsc.subcore_barrier()
    flush(sum_a)
    if with_counts:
        @pl.when(s == 0)
        def _():
            pltpu.sync_copy(cacc, cnt_hbm)
    zero_acc()
    plsc.subcore_barrier()
    accumulate(h_b, False)
    plsc.subcore_barrier()
    flush(sum_b)


def _scatter_kernel_body(row_hbm, h0, h1, h2, h3, z_hbm, zc_hbm,
                         s0, s1, s2, s3, cnt_hbm,
                         idx_v, upd_v, ones_v, acc, cacc, sem):
    c = lax.axis_index("c")
    s = lax.axis_index("s")
    nb = (EPAD // NS) // IB
    pltpu.sync_copy(row_hbm.at[pl.ds(s * nb, nb)], idx_v)

    def ones_init(i, _):
        ones_v[pl.ds(i * L, L)] = jnp.full((L,), 1.0, jnp.float32)
        return 0

    lax.fori_loop(0, IB // L, ones_init, 0)

    @pl.when(c == 0)
    def _():
        _scatter_chunks(s, h0, h1, s0, s1, z_hbm, zc_hbm, cnt_hbm,
                        idx_v, upd_v, ones_v, acc, cacc, sem, True)

    @pl.when(c == 1)
    def _():
        _scatter_chunks(s, h2, h3, s2, s3, z_hbm, zc_hbm, cnt_hbm,
                        idx_v, upd_v, ones_v, acc, cacc, sem, False)


@functools.lru_cache(maxsize=None)
def _build_sc_scatter():
    mesh = plsc.VectorSubcoreMesh(core_axis_name="c", subcore_axis_name="s",
                                  num_cores=NC, num_subcores=NS)
    return pl.kernel(
        _scatter_kernel_body,
        out_type=[jax.ShapeDtypeStruct((NPAD, 128), jnp.float32)] * NCHUNK
                 + [jax.ShapeDtypeStruct((NPAD,), jnp.float32)],
        mesh=mesh,
        scratch_types=[
            pltpu.VMEM(((EPAD // NS) // IB, IB), jnp.int32),
            pltpu.VMEM((2, IB, 128), jnp.float32),
            pltpu.VMEM((IB,), jnp.float32),
            pltpu.VMEM_SHARED((NPAD, 128), jnp.float32),
            pltpu.VMEM_SHARED((NPAD,), jnp.float32),
            pltpu.SemaphoreType.DMA((2,)),
        ],
    )


def _sc_scatter(row_pad, h0, h1, h2, h3, zeros_h, zeros_c):
    return _build_sc_scatter()(row_pad, h0, h1, h2, h3, zeros_h, zeros_c)


BE = 256


def _edge_mlp_body(xg_ref, e_ref, w1a_ref, w1b_ref, b1_ref,
                   h0_ref, h1_ref, h2_ref, h3_ref):
    h = jnp.dot(xg_ref[...], w1a_ref[...], preferred_element_type=jnp.float32)
    h = h + jnp.dot(e_ref[...], w1b_ref[...],
                    preferred_element_type=jnp.float32)
    h = jnp.maximum(h + b1_ref[...], 0.0)
    h0_ref[...] = h[:, 0:128]
    h1_ref[...] = h[:, 128:256]
    h2_ref[...] = h[:, 256:384]
    h3_ref[...] = h[:, 384:512]


def _edge_mlp(xg, e, w1a, w1b, b1):
    grid = (NE // BE,)
    return pl.pallas_call(
        _edge_mlp_body,
        grid=grid,
        in_specs=[
            pl.BlockSpec((BE, 128), lambda i: (i, 0)),
            pl.BlockSpec((BE, H), lambda i: (i, 0)),
            pl.BlockSpec((128, H), lambda i: (0, 0)),
            pl.BlockSpec((H, H), lambda i: (0, 0)),
            pl.BlockSpec((1, H), lambda i: (0, 0)),
        ],
        out_specs=[pl.BlockSpec((BE, 128), lambda i: (i, 0))] * NCHUNK,
        out_shape=[jax.ShapeDtypeStruct((EPAD, 128), jnp.float32)] * NCHUNK,
    )(xg, e, w1a, w1b, b1)


BN = 256


def _node_mlp_body(xp_ref, s0_ref, s1_ref, s2_ref, s3_ref, cnt_ref, b2d_ref,
                   u_ref, w2_ref, b2_ref, w3a_ref, w3b_ref, w3c_ref, b3_ref,
                   w4_ref, b4_ref, o_ref):
    cnt = cnt_ref[...]
    m = jnp.maximum(cnt, 1.0)
    meanh = jnp.concatenate(
        [s0_ref[...], s1_ref[...], s2_ref[...], s3_ref[...]], axis=1) / m
    agg = jnp.dot(meanh, w2_ref[...], preferred_element_type=jnp.float32)
    agg = agg + jnp.where(cnt > 0.0, 1.0, 0.0) * b2_ref[...]
    hid = jnp.dot(xp_ref[...], w3a_ref[...],
                  preferred_element_type=jnp.float32)
    hid = hid + jnp.dot(agg, w3b_ref[...], preferred_element_type=jnp.float32)
    onehot = (b2d_ref[...] == lax.broadcasted_iota(
        jnp.int32, (BN, 64), 1)).astype(jnp.float32)
    ug = jnp.dot(onehot, u_ref[...], preferred_element_type=jnp.float32)
    hid = hid + jnp.dot(ug, w3c_ref[...], preferred_element_type=jnp.float32)
    hid = jnp.maximum(hid + b3_ref[...], 0.0)
    o_ref[...] = jnp.dot(hid, w4_ref[...],
                         preferred_element_type=jnp.float32) + b4_ref[...]


def _node_mlp(xp, s0, s1, s2, s3, cnt, b2d, u, w2, b2, w3a, w3b, w3c, b3,
              w4, b4):
    grid = (NPAD // BN,)
    return pl.pallas_call(
        _node_mlp_body,
        grid=grid,
        in_specs=[
            pl.BlockSpec((BN, 128), lambda i: (i, 0)),
            pl.BlockSpec((BN, 128), lambda i: (i, 0)),
            pl.BlockSpec((BN, 128), lambda i: (i, 0)),
            pl.BlockSpec((BN, 128), lambda i: (i, 0)),
            pl.BlockSpec((BN, 128), lambda i: (i, 0)),
            pl.BlockSpec((BN, 1), lambda i: (i, 0)),
            pl.BlockSpec((BN, 1), lambda i: (i, 0)),
            pl.BlockSpec((64, 16), lambda i: (0, 0)),
            pl.BlockSpec((H, H), lambda i: (0, 0)),
            pl.BlockSpec((1, H), lambda i: (0, 0)),
            pl.BlockSpec((128, H), lambda i: (0, 0)),
            pl.BlockSpec((H, H), lambda i: (0, 0)),
            pl.BlockSpec((16, H), lambda i: (0, 0)),
            pl.BlockSpec((1, H), lambda i: (0, 0)),
            pl.BlockSpec((H, 1), lambda i: (0, 0)),
            pl.BlockSpec((1, 1), lambda i: (0, 0)),
        ],
        out_specs=pl.BlockSpec((BN, 1), lambda i: (i, 0)),
        out_shape=jax.ShapeDtypeStruct((NPAD, 1), jnp.float32),
    )(xp, s0, s1, s2, s3, cnt, b2d, u, w2, b2, w3a, w3b, w3c, b3, w4, b4)


def kernel(x, edge_index, edge_attr, u, batch, W1, b1, W2, b2, W3, b3, W4,
           b4):
    f32 = jnp.float32
    row = edge_index[0].astype(jnp.int32)
    col = edge_index[1].astype(jnp.int32)

    x_pad = jnp.zeros((NPAD, 128), f32).at[:N, :9].set(x)
    col_fill = jnp.arange(EPAD - NE, dtype=jnp.int32) * 41 % N
    col_pad = jnp.concatenate([col, col_fill]).reshape(EPAD // IB, IB)
    sent = (jnp.arange(EPAD - NE, dtype=jnp.int32) % (NPAD - N)) + N
    row_pad = jnp.concatenate([row, sent]).reshape(EPAD // IB, IB)

    w1a = jnp.zeros((128, H), f32).at[:9].set(W1[:9])
    w1b = W1[9:]
    w3a = jnp.zeros((128, H), f32).at[:9].set(W3[:9])
    w3b = W3[9:9 + H]
    w3c = W3[9 + H:]
    batch2d = jnp.concatenate(
        [batch.astype(jnp.int32),
         jnp.zeros((NPAD - N,), jnp.int32)]).reshape(NPAD, 1)
    zeros_h = jnp.zeros((NPAD, 128), f32)
    zeros_c = jnp.zeros((NPAD,), f32)

    xg = _sc_gather(x_pad, col_pad)
    h0, h1, h2, h3 = _edge_mlp(xg, edge_attr, w1a, w1b, b1.reshape(1, H))
    s0, s1, s2, s3, cnt = _sc_scatter(row_pad, h0, h1, h2, h3, zeros_h,
                                      zeros_c)
    out = _node_mlp(x_pad, s0, s1, s2, s3, cnt.reshape(NPAD, 1), batch2d, u,
                    W2, b2.reshape(1, H), w3a, w3b, w3c, b3.reshape(1, H),
                    W4, b4.reshape(1, 1))
    return out[:N]


# speedup vs baseline: 2.9739x; 1.2234x over previous
import functools

import jax
import jax.numpy as jnp
from jax import lax
from jax.experimental import pallas as pl
from jax.experimental.pallas import tpu as pltpu
from jax.experimental.pallas import tpu_sc as plsc

N = 10000
NE = 160000
H = 512
NPAD = 10240
EPAD = 163840
NC, NS, L = 2, 16, 16
IB = 128
NCHUNK = 4

def _gather_body(x_hbm, idx_hbm, out_hbm, idx_v, buf_v, sem):
    c = lax.axis_index("c")
    s = lax.axis_index("s")
    wid = s * NC + c
    ew = EPAD // (NC * NS)
    nb = ew // IB
    pltpu.sync_copy(idx_hbm.at[pl.ds(wid * nb, nb)], idx_v)

    buf0, buf1 = buf_v.at[0], buf_v.at[1]
    sem0, sem1 = sem.at[0], sem.at[1]
    pltpu.async_copy(x_hbm.at[idx_v.at[0]], buf0, sem0)

    def body(jj, _):
        j = 2 * jj
        pltpu.async_copy(x_hbm.at[idx_v.at[j + 1]], buf1, sem1)
        pltpu.make_async_copy(x_hbm.at[idx_v.at[j]], buf0, sem0).wait()
        pltpu.sync_copy(buf0, out_hbm.at[pl.ds(wid * ew + j * IB, IB)])

        @pl.when(jj < nb // 2 - 1)
        def _():
            pltpu.async_copy(x_hbm.at[idx_v.at[j + 2]], buf0, sem0)

        pltpu.make_async_copy(x_hbm.at[idx_v.at[j + 1]], buf1, sem1).wait()
        pltpu.sync_copy(buf1, out_hbm.at[pl.ds(wid * ew + (j + 1) * IB, IB)])
        return 0

    lax.fori_loop(0, nb // 2, body, 0)


@functools.lru_cache(maxsize=None)
def _build_sc_gather():
    mesh = plsc.VectorSubcoreMesh(core_axis_name="c", subcore_axis_name="s",
                                  num_cores=NC, num_subcores=NS)
    return pl.kernel(
        _gather_body,
        out_type=jax.ShapeDtypeStruct((EPAD, 128), jnp.float32),
        mesh=mesh,
        scratch_types=[
            pltpu.VMEM((EPAD // (NC * NS) // IB, IB), jnp.int32),
            pltpu.VMEM((2, IB, 128), jnp.float32),
            pltpu.SemaphoreType.DMA((2,)),
        ],
    )


def _sc_gather(x_pad, col_pad):
    return _build_sc_gather()(x_pad, col_pad)


def _scatter_chunks(s, h_a, h_b, sum_a, sum_b, z_hbm, zc_hbm, cnt_hbm,
                    idx_v, upd_v, ones_v, acc, cacc, sem, with_counts):
    nb = (EPAD // NS) // IB
    stripe = NPAD // NS

    def zero_acc():
        pltpu.sync_copy(z_hbm.at[pl.ds(s * stripe, stripe)],
                        acc.at[pl.ds(s * stripe, stripe)])

    def accumulate(h_ref, counts):
        base = s * (EPAD // NS)
        upd0, upd1 = upd_v.at[0], upd_v.at[1]
        sem0, sem1 = sem.at[0], sem.at[1]
        pltpu.async_copy(h_ref.at[pl.ds(base, IB)], upd0, sem0)

        def body(jj, _):
            j = 2 * jj
            pltpu.async_copy(h_ref.at[pl.ds(base + (j + 1) * IB, IB)],
                             upd1, sem1)
            pltpu.make_async_copy(h_ref.at[pl.ds(base + j * IB, IB)],
                                  upd0, sem0).wait()
            pltpu.sync_copy(upd0, acc.at[idx_v.at[j]], add=True)
            if counts:
                pltpu.sync_copy(ones_v, cacc.at[idx_v.at[j]], add=True)

            @pl.when(jj < nb // 2 - 1)
            def _():
                pltpu.async_copy(h_ref.at[pl.ds(base + (j + 2) * IB, IB)],
                                 upd0, sem0)

            pltpu.make_async_copy(h_ref.at[pl.ds(base + (j + 1) * IB, IB)],
                                  upd1, sem1).wait()
            pltpu.sync_copy(upd1, acc.at[idx_v.at[j + 1]], add=True)
            if counts:
                pltpu.sync_copy(ones_v, cacc.at[idx_v.at[j + 1]], add=True)
            return 0

        lax.fori_loop(0, nb // 2, body, 0)

    def flush(sum_ref):
        pltpu.sync_copy(acc.at[pl.ds(s * stripe, stripe)],
                        sum_ref.at[pl.ds(s * stripe, stripe)])

    zero_acc()
    if with_counts:
        @pl.when(s == 0)
        def _():
            pltpu.sync_copy(zc_hbm, cacc)
    plsc.subcore_barrier()
    accumulate(h_a, with_counts)
    plsc.subcore_barrier()
    flush(sum_a)
    if with_counts:
        @pl.when(s == 0)
        def _():
            pltpu.sync_copy(cacc, cnt_hbm)
    zero_acc()
    plsc.subcore_barrier()
    accumulate(h_b, False)
    plsc.subcore_barrier()
    flush(sum_b)


def _scatter_kernel_body(row_hbm, h0, h1, h2, h3, z_hbm, zc_hbm,
                         s0, s1, s2, s3, cnt_hbm,
                         idx_v, upd_v, ones_v, acc, cacc, sem):
    c = lax.axis_index("c")
    s = lax.axis_index("s")
    nb = (EPAD // NS) // IB
    pltpu.sync_copy(row_hbm.at[pl.ds(s * nb, nb)], idx_v)

    def ones_init(i, _):
        ones_v[pl.ds(i * L, L)] = jnp.full((L,), 1.0, jnp.float32)
        return 0

    lax.fori_loop(0, IB // L, ones_init, 0)

    @pl.when(c == 0)
    def _():
        _scatter_chunks(s, h0, h1, s0, s1, z_hbm, zc_hbm, cnt_hbm,
                        idx_v, upd_v, ones_v, acc, cacc, sem, True)

    @pl.when(c == 1)
    def _():
        _scatter_chunks(s, h2, h3, s2, s3, z_hbm, zc_hbm, cnt_hbm,
                        idx_v, upd_v, ones_v, acc, cacc, sem, False)


@functools.lru_cache(maxsize=None)
def _build_sc_scatter():
    mesh = plsc.VectorSubcoreMesh(core_axis_name="c", subcore_axis_name="s",
                                  num_cores=NC, num_subcores=NS)
    return pl.kernel(
        _scatter_kernel_body,
        out_type=[jax.ShapeDtypeStruct((NPAD, 128), jnp.float32)] * NCHUNK
                 + [jax.ShapeDtypeStruct((NPAD,), jnp.float32)],
        mesh=mesh,
        scratch_types=[
            pltpu.VMEM(((EPAD // NS) // IB, IB), jnp.int32),
            pltpu.VMEM((2, IB, 128), jnp.float32),
            pltpu.VMEM((IB,), jnp.float32),
            pltpu.VMEM_SHARED((NPAD, 128), jnp.float32),
            pltpu.VMEM_SHARED((NPAD,), jnp.float32),
            pltpu.SemaphoreType.DMA((2,)),
        ],
    )


def _sc_scatter(row_pad, h0, h1, h2, h3, zeros_h, zeros_c):
    return _build_sc_scatter()(row_pad, h0, h1, h2, h3, zeros_h, zeros_c)


BE = 512


def _edge_mlp_body(xg_ref, e_ref, w1a_ref, w1b_ref, b1_ref,
                   h0_ref, h1_ref, h2_ref, h3_ref):
    bf = jnp.bfloat16
    h = jnp.dot(xg_ref[...].astype(bf), w1a_ref[...].astype(bf),
                preferred_element_type=jnp.float32)
    h = h + jnp.dot(e_ref[...].astype(bf), w1b_ref[...].astype(bf),
                    preferred_element_type=jnp.float32)
    h = jnp.maximum(h + b1_ref[...], 0.0)
    h0_ref[...] = h[:, 0:128]
    h1_ref[...] = h[:, 128:256]
    h2_ref[...] = h[:, 256:384]
    h3_ref[...] = h[:, 384:512]


def _edge_mlp(xg, e, w1a, w1b, b1):
    grid = (NE // BE,)
    return pl.pallas_call(
        _edge_mlp_body,
        grid=grid,
        in_specs=[
            pl.BlockSpec((BE, 128), lambda i: (i, 0)),
            pl.BlockSpec((BE, H), lambda i: (i, 0)),
            pl.BlockSpec((128, H), lambda i: (0, 0)),
            pl.BlockSpec((H, H), lambda i: (0, 0)),
            pl.BlockSpec((1, H), lambda i: (0, 0)),
        ],
        out_specs=[pl.BlockSpec((BE, 128), lambda i: (i, 0))] * NCHUNK,
        out_shape=[jax.ShapeDtypeStruct((EPAD, 128), jnp.float32)] * NCHUNK,
    )(xg, e, w1a, w1b, b1)


BN = 256


def _node_mlp_body(xp_ref, s0_ref, s1_ref, s2_ref, s3_ref, cnt_ref, b2d_ref,
                   u_ref, w2_ref, b2_ref, w3a_ref, w3b_ref, w3c_ref, b3_ref,
                   w4_ref, b4_ref, o_ref):
    cnt = cnt_ref[...]
    m = jnp.maximum(cnt, 1.0)
    meanh = jnp.concatenate(
        [s0_ref[...], s1_ref[...], s2_ref[...], s3_ref[...]], axis=1) / m
    agg = jnp.dot(meanh, w2_ref[...], preferred_element_type=jnp.float32)
    agg = agg + jnp.where(cnt > 0.0, 1.0, 0.0) * b2_ref[...]
    hid = jnp.dot(xp_ref[...], w3a_ref[...],
                  preferred_element_type=jnp.float32)
    hid = hid + jnp.dot(agg, w3b_ref[...], preferred_element_type=jnp.float32)
    onehot = (b2d_ref[...] == lax.broadcasted_iota(
        jnp.int32, (BN, 64), 1)).astype(jnp.float32)
    ug = jnp.dot(onehot, u_ref[...], preferred_element_type=jnp.float32)
    hid = hid + jnp.dot(ug, w3c_ref[...], preferred_element_type=jnp.float32)
    hid = jnp.maximum(hid + b3_ref[...], 0.0)
    o_ref[...] = jnp.dot(hid, w4_ref[...],
                         preferred_element_type=jnp.float32) + b4_ref[...]


def _node_mlp(xp, s0, s1, s2, s3, cnt, b2d, u, w2, b2, w3a, w3b, w3c, b3,
              w4, b4):
    grid = (NPAD // BN,)
    return pl.pallas_call(
        _node_mlp_body,
        grid=grid,
        in_specs=[
            pl.BlockSpec((BN, 128), lambda i: (i, 0)),
            pl.BlockSpec((BN, 128), lambda i: (i, 0)),
            pl.BlockSpec((BN, 128), lambda i: (i, 0)),
            pl.BlockSpec((BN, 128), lambda i: (i, 0)),
            pl.BlockSpec((BN, 128), lambda i: (i, 0)),
            pl.BlockSpec((BN, 1), lambda i: (i, 0)),
            pl.BlockSpec((BN, 1), lambda i: (i, 0)),
            pl.BlockSpec((64, 16), lambda i: (0, 0)),
            pl.BlockSpec((H, H), lambda i: (0, 0)),
            pl.BlockSpec((1, H), lambda i: (0, 0)),
            pl.BlockSpec((128, H), lambda i: (0, 0)),
            pl.BlockSpec((H, H), lambda i: (0, 0)),
            pl.BlockSpec((16, H), lambda i: (0, 0)),
            pl.BlockSpec((1, H), lambda i: (0, 0)),
            pl.BlockSpec((H, 1), lambda i: (0, 0)),
            pl.BlockSpec((1, 1), lambda i: (0, 0)),
        ],
        out_specs=pl.BlockSpec((BN, 1), lambda i: (i, 0)),
        out_shape=jax.ShapeDtypeStruct((NPAD, 1), jnp.float32),
    )(xp, s0, s1, s2, s3, cnt, b2d, u, w2, b2, w3a, w3b, w3c, b3, w4, b4)


def kernel(x, edge_index, edge_attr, u, batch, W1, b1, W2, b2, W3, b3, W4,
           b4):
    f32 = jnp.float32
    row = edge_index[0].astype(jnp.int32)
    col = edge_index[1].astype(jnp.int32)

    x_pad = jnp.zeros((NPAD, 128), f32).at[:N, :9].set(x)
    col_fill = jnp.arange(EPAD - NE, dtype=jnp.int32) * 41 % N
    col_pad = jnp.concatenate([col, col_fill]).reshape(EPAD // IB, IB)
    sent = (jnp.arange(EPAD - NE, dtype=jnp.int32) % (NPAD - N)) + N
    row_pad = jnp.concatenate([row, sent]).reshape(EPAD // IB, IB)

    w1a = jnp.zeros((128, H), f32).at[:9].set(W1[:9])
    w1b = W1[9:]
    w3a = jnp.zeros((128, H), f32).at[:9].set(W3[:9])
    w3b = W3[9:9 + H]
    w3c = W3[9 + H:]
    batch2d = jnp.concatenate(
        [batch.astype(jnp.int32),
         jnp.zeros((NPAD - N,), jnp.int32)]).reshape(NPAD, 1)
    zeros_h = jnp.zeros((NPAD, 128), f32)
    zeros_c = jnp.zeros((NPAD,), f32)

    xg = _sc_gather(x_pad, col_pad)
    h0, h1, h2, h3 = _edge_mlp(xg, edge_attr, w1a, w1b, b1.reshape(1, H))
    s0, s1, s2, s3, cnt = _sc_scatter(row_pad, h0, h1, h2, h3, zeros_h,
                                      zeros_c)
    out = _node_mlp(x_pad, s0, s1, s2, s3, cnt.reshape(NPAD, 1), batch2d, u,
                    W2, b2.reshape(1, H), w3a, w3b, w3c, b3.reshape(1, H),
                    W4, b4.reshape(1, 1))
    return out[:N]
